# Initial kernel scaffold; baseline (speedup 1.0000x reference)
import jax
import jax.numpy as jnp
from jax.experimental import pallas as pl


def kernel(input_ids, table):
    raise NotImplementedError("write your pallas kernel here")


# speedup vs baseline: 1.8446x; 1.8446x over previous
import functools

import jax
import jax.numpy as jnp
from jax import lax
from jax.experimental import pallas as pl
from jax.experimental.pallas import tpu as pltpu
from jax.experimental.pallas import tpu_sc as plsc

_LANES = 128
_CHUNK = 1024
_KSUB = _CHUNK // _LANES


def _emb_lookup(idx2d, table, n_rows):
    info = plsc.get_sparse_core_info()
    nw = info.num_cores * info.num_subcores
    b_per_w = n_rows // nw
    n_chunks = b_per_w // _CHUNK
    dim = table.shape[1]
    mesh = plsc.VectorSubcoreMesh(core_axis_name="c", subcore_axis_name="s")

    @functools.partial(
        pl.kernel,
        mesh=mesh,
        compiler_params=pltpu.CompilerParams(use_tc_tiling_on_sc=False),
        out_type=jax.ShapeDtypeStruct((n_rows, dim), jnp.float32),
        scratch_types=[
            pltpu.VMEM((_KSUB, _LANES), jnp.int32),
            pltpu.VMEM((_CHUNK, dim), jnp.float32),
            pltpu.SemaphoreType.DMA,
        ],
    )
    def emb(idx_hbm, tab_hbm, out_hbm, idx_v, rows_v, sem):
        wid = lax.axis_index("s") * info.num_cores + lax.axis_index("c")
        base = wid * b_per_w

        def body(i, carry):
            off = base + i * _CHUNK
            row_off = pl.multiple_of(off // _LANES, 8)
            pltpu.sync_copy(idx_hbm.at[pl.ds(row_off, _KSUB)], idx_v)
            copies = [
                pltpu.async_copy(
                    tab_hbm.at[idx_v.at[j]],
                    rows_v.at[pl.ds(j * _LANES, _LANES)],
                    sem,
                )
                for j in range(_KSUB)
            ]
            for c in copies:
                c.wait()
            pltpu.sync_copy(rows_v, out_hbm.at[pl.ds(off, _CHUNK)])
            return carry

        lax.fori_loop(0, n_chunks, body, 0)

    return emb(idx2d, table)


def kernel(input_ids, table):
    b, h = input_ids.shape
    n = b * h
    idx2d = input_ids.reshape(n // _LANES, _LANES).astype(jnp.int32)
    out = _emb_lookup(idx2d, table, n)
    return out.reshape(b, h, table.shape[1])


# speedup vs baseline: 1.8647x; 1.0109x over previous
import functools

import jax
import jax.numpy as jnp
from jax import lax
from jax.experimental import pallas as pl
from jax.experimental.pallas import tpu as pltpu
from jax.experimental.pallas import tpu_sc as plsc

_LANES = 128
_CHUNK = 512
_KSUB = _CHUNK // _LANES


def _emb_lookup(idx2d, table, n_rows):
    info = plsc.get_sparse_core_info()
    nw = info.num_cores * info.num_subcores
    b_per_w = n_rows // nw
    n_chunks = b_per_w // _CHUNK
    idx_rows = b_per_w // _LANES
    dim = table.shape[1]
    mesh = plsc.VectorSubcoreMesh(core_axis_name="c", subcore_axis_name="s")

    @functools.partial(
        pl.kernel,
        mesh=mesh,
        compiler_params=pltpu.CompilerParams(use_tc_tiling_on_sc=False),
        out_type=jax.ShapeDtypeStruct((n_rows, dim), jnp.float32),
        scratch_types=[
            pltpu.VMEM((idx_rows, _LANES), jnp.int32),
            pltpu.VMEM((_CHUNK, dim), jnp.float32),
            pltpu.VMEM((_CHUNK, dim), jnp.float32),
            pltpu.SemaphoreType.DMA,
            pltpu.SemaphoreType.DMA,
            pltpu.SemaphoreType.DMA,
            pltpu.SemaphoreType.DMA,
        ],
    )
    def emb(idx_hbm, tab_hbm, out_hbm, idx_all, rows0, rows1,
            gsem0, gsem1, osem0, osem1):
        wid = lax.axis_index("s") * info.num_cores + lax.axis_index("c")
        base = wid * b_per_w
        rows = (rows0, rows1)
        gsem = (gsem0, gsem1)
        osem = (osem0, osem1)

        pltpu.sync_copy(
            idx_hbm.at[pl.ds(pl.multiple_of(base // _LANES, 8), idx_rows)],
            idx_all)

        def fire(chunk, slot):
            return [
                pltpu.async_copy(
                    tab_hbm.at[idx_all.at[chunk * _KSUB + j]],
                    rows[slot].at[pl.ds(j * _LANES, _LANES)],
                    gsem[slot],
                )
                for j in range(_KSUB)
            ]

        def start_out(chunk, slot):
            return pltpu.async_copy(
                rows[slot],
                out_hbm.at[pl.ds(base + chunk * _CHUNK, _CHUNK)],
                osem[slot],
            )

        def drain_out(slot):
            pltpu.make_async_copy(
                rows[slot],
                out_hbm.at[pl.ds(base, _CHUNK)],
                osem[slot],
            ).wait()

        def body(k, carry):
            a = 2 * k
            b = a + 1

            @pl.when(k > 0)
            def _():
                drain_out(0)

            ga = fire(a, 0)

            @pl.when(k > 0)
            def _():
                drain_out(1)

            gb = fire(b, 1)
            for c in ga:
                c.wait()
            start_out(a, 0)
            for c in gb:
                c.wait()
            start_out(b, 1)
            return carry

        lax.fori_loop(0, n_chunks // 2, body, 0)
        drain_out(0)
        drain_out(1)

    return emb(idx2d, table)


def kernel(input_ids, table):
    b, h = input_ids.shape
    n = b * h
    idx2d = input_ids.reshape(n // _LANES, _LANES).astype(jnp.int32)
    out = _emb_lookup(idx2d, table, n)
    return out.reshape(b, h, table.shape[1])
